# baseline (speedup 1.0000x reference)
import functools

import jax
import jax.numpy as jnp
from jax import lax
from jax.experimental import pallas as pl
from jax.experimental.pallas import tpu as pltpu
from jax.experimental.pallas import tpu_sc as plsc

_NC = 2
_NS = 16
_LANES = 16
_ROW = 128


@functools.lru_cache(maxsize=None)
def _build_gather(B, V):
    NW = _NC * _NS
    assert B % (8 * NW) == 0
    b_per_w = B // NW
    mesh = plsc.VectorSubcoreMesh(core_axis_name="c", subcore_axis_name="s")

    @functools.partial(
        pl.kernel,
        mesh=mesh,
        out_type=jax.ShapeDtypeStruct((B, _ROW), jnp.float32),
        scratch_types=[
            pltpu.VMEM((b_per_w,), jnp.int32),
            pltpu.VMEM((b_per_w, _ROW), jnp.float32),
            pltpu.SemaphoreType.DMA,
        ],
    )
    def k(idx_hbm, table_hbm, out_hbm, idx_v, rows_v, sem):
        c = lax.axis_index("c")
        s = lax.axis_index("s")
        wid = s * _NC + c
        base = wid * b_per_w

        pltpu.sync_copy(idx_hbm.at[pl.ds(base, b_per_w)], idx_v)
        pltpu.async_copy(table_hbm.at[idx_v], rows_v, sem).wait()
        pltpu.sync_copy(rows_v, out_hbm.at[pl.ds(base, b_per_w)])

    return k


def _count_body(l_ref, out_ref):
    out_ref[0] = jnp.sum((l_ref[...] > 0).astype(jnp.int32))


@functools.lru_cache(maxsize=None)
def _build_count(B):
    return pl.pallas_call(
        _count_body,
        out_shape=jax.ShapeDtypeStruct((1,), jnp.int32),
        in_specs=[pl.BlockSpec(memory_space=pltpu.VMEM)],
        out_specs=pl.BlockSpec(memory_space=pltpu.SMEM),
    )


def kernel(X, L, W):
    B = X.shape[0]
    V, D = W.shape
    idx = X[:, 0]
    table = jnp.pad(W, ((0, 0), (0, _ROW - D)))
    padded = _build_gather(B, V)(idx, table)
    data = padded[:, :D]
    bs = _build_count(B)(L.astype(jnp.int32))
    return data, bs


# speedup vs baseline: 1.0583x; 1.0583x over previous
import functools

import jax
import jax.numpy as jnp
from jax import lax
from jax.experimental import pallas as pl
from jax.experimental.pallas import tpu as pltpu
from jax.experimental.pallas import tpu_sc as plsc

_NC = 2
_NS = 16
_LANES = 16


@functools.lru_cache(maxsize=None)
def _build_gather(B, D, V):
    NW = _NC * _NS
    assert B % (8 * NW) == 0 and D % _LANES == 0
    b_per_w = B // NW
    mesh = plsc.VectorSubcoreMesh(core_axis_name="c", subcore_axis_name="s")

    @functools.partial(
        pl.kernel,
        mesh=mesh,
        compiler_params=pltpu.CompilerParams(use_tc_tiling_on_sc=False),
        out_type=jax.ShapeDtypeStruct((B, D), jnp.float32),
        scratch_types=[
            pltpu.VMEM((b_per_w,), jnp.int32),
            pltpu.VMEM((b_per_w, D), jnp.float32),
            pltpu.SemaphoreType.DMA,
            pltpu.SemaphoreType.DMA,
        ],
    )
    def k(idx_hbm, table_hbm, out_hbm, idx_v, rows_v, sem, sem2):
        c = lax.axis_index("c")
        s = lax.axis_index("s")
        wid = s * _NC + c
        base = wid * b_per_w

        h = b_per_w // 2
        pltpu.sync_copy(idx_hbm.at[pl.ds(base, b_per_w)], idx_v)
        g0 = pltpu.async_copy(
            table_hbm.at[idx_v.at[pl.ds(0, h)]], rows_v.at[pl.ds(0, h)], sem)
        g1 = pltpu.async_copy(
            table_hbm.at[idx_v.at[pl.ds(h, h)]], rows_v.at[pl.ds(h, h)], sem2)
        g0.wait()
        pltpu.sync_copy(rows_v.at[pl.ds(0, h)], out_hbm.at[pl.ds(base, h)])
        g1.wait()
        pltpu.sync_copy(rows_v.at[pl.ds(h, h)], out_hbm.at[pl.ds(base + h, h)])

    return k


def _count_body(l_ref, out_ref):
    out_ref[0] = jnp.sum((l_ref[...] > 0).astype(jnp.int32))


@functools.lru_cache(maxsize=None)
def _build_count(B):
    return pl.pallas_call(
        _count_body,
        out_shape=jax.ShapeDtypeStruct((1,), jnp.int32),
        in_specs=[pl.BlockSpec(memory_space=pltpu.VMEM)],
        out_specs=pl.BlockSpec(memory_space=pltpu.SMEM),
    )


def kernel(X, L, W):
    B = X.shape[0]
    V, D = W.shape
    idx = X[:, 0]
    data = _build_gather(B, D, V)(idx, W)
    bs = _build_count(B)(L.astype(jnp.int32))
    return data, bs


# speedup vs baseline: 1.1120x; 1.0507x over previous
import functools

import jax
import jax.numpy as jnp
from jax import lax
from jax.experimental import pallas as pl
from jax.experimental.pallas import tpu as pltpu
from jax.experimental.pallas import tpu_sc as plsc

_NS = 16
_LANES = 16


@functools.lru_cache(maxsize=None)
def _build_gather(B, D, V, nc=1):
    NW = nc * _NS
    assert B % (8 * NW) == 0 and D % _LANES == 0
    b_per_w = B // NW
    n_g = -(-b_per_w // 128)
    g = b_per_w // n_g
    mesh = plsc.VectorSubcoreMesh(
        core_axis_name="c", subcore_axis_name="s", num_cores=nc)

    @functools.partial(
        pl.kernel,
        mesh=mesh,
        compiler_params=pltpu.CompilerParams(use_tc_tiling_on_sc=False),
        out_type=jax.ShapeDtypeStruct((B, D), jnp.float32),
        scratch_types=[
            pltpu.VMEM((b_per_w,), jnp.int32),
            pltpu.VMEM((b_per_w, D), jnp.float32),
            pltpu.SemaphoreType.DMA,
        ],
    )
    def k(idx_hbm, table_hbm, out_hbm, idx_v, rows_v, sem):
        c = lax.axis_index("c")
        s = lax.axis_index("s")
        wid = s * nc + c
        base = wid * b_per_w

        pltpu.sync_copy(idx_hbm.at[pl.ds(base, b_per_w)], idx_v)
        gs = [
            pltpu.async_copy(
                table_hbm.at[idx_v.at[pl.ds(i * g, g)]],
                rows_v.at[pl.ds(i * g, g)], sem)
            for i in range(n_g)
        ]
        for i, gg in enumerate(gs):
            gg.wait()
            pltpu.sync_copy(rows_v.at[pl.ds(i * g, g)],
                            out_hbm.at[pl.ds(base + i * g, g)])

    return k


def _count_body(l_ref, out_ref):
    out_ref[0] = jnp.sum((l_ref[...] > 0).astype(jnp.int32))


@functools.lru_cache(maxsize=None)
def _build_count(B):
    return pl.pallas_call(
        _count_body,
        out_shape=jax.ShapeDtypeStruct((1,), jnp.int32),
        in_specs=[pl.BlockSpec(memory_space=pltpu.VMEM)],
        out_specs=pl.BlockSpec(memory_space=pltpu.SMEM),
    )


def kernel(X, L, W):
    B = X.shape[0]
    V, D = W.shape
    idx = X[:, 0]
    data = _build_gather(B, D, V)(idx, W)
    bs = _build_count(B)(L.astype(jnp.int32))
    return data, bs
